# baseline (device time: 264617 ns/iter reference)
import jax
import jax.numpy as jnp
from jax import lax
from jax.experimental import pallas as pl
from jax.experimental.pallas import tpu as pltpu

B, S, H_LOC, D = 4, 1024, 16, 128
K = H_LOC * D
N = 4096
S_HALF = S // 2
NC = 512
G = N // NC
ROWS = B * S_HALF


def kernel(O, Wo):
    O2 = O.reshape(B * S, K).astype(jnp.bfloat16)
    Wo2 = Wo.astype(jnp.bfloat16)

    def body(o_ref, wo_hbm, out_ref, wo_buf, send_buf, recv_buf, acc_buf,
             wo_sem, out_sem, send_sems, recv_sems):
        my_x = lax.axis_index("x")
        my_y = lax.axis_index("y")
        my_z = lax.axis_index("z")
        peer = (my_x, 1 - my_y, my_z)

        barrier = pltpu.get_barrier_semaphore()
        pl.semaphore_signal(barrier, inc=1, device_id=peer,
                            device_id_type=pl.DeviceIdType.MESH)
        pl.semaphore_wait(barrier, 1)

        my_base = my_y * S_HALF
        peer_base = (1 - my_y) * S_HALF

        def wo_fetch(j):
            return pltpu.make_async_copy(
                wo_hbm.at[:, pl.ds((j % G) * NC, NC)], wo_buf.at[j % 2],
                wo_sem.at[j % 2])

        rdmas = [None] * G
        ocps = [None] * G

        wo_fetch(0).start()
        for j in range(G):
            if j + 1 < G:
                wo_fetch(j + 1).start()
            wo_fetch(j).wait()
            if j >= 4:
                rdmas[j - 4].wait_send()
            for b in range(B):
                lhs = o_ref[pl.ds(b * S + peer_base, S_HALF), :]
                send_buf[j % 4, b * S_HALF:(b + 1) * S_HALF, :] = jnp.dot(
                    lhs, wo_buf[j % 2], preferred_element_type=jnp.float32
                ).astype(jnp.bfloat16)
            rdmas[j] = pltpu.make_async_remote_copy(
                src_ref=send_buf.at[j % 4],
                dst_ref=recv_buf.at[j],
                send_sem=send_sems.at[j % 4],
                recv_sem=recv_sems.at[j],
                device_id=peer,
                device_id_type=pl.DeviceIdType.MESH,
            )
            rdmas[j].start()

        wo_fetch(G).start()
        for j in range(G):
            if j + 1 < G:
                wo_fetch(G + j + 1).start()
            wo_fetch(G + j).wait()
            if j >= 2:
                ocps[j - 2].wait()
            rdmas[j].wait_recv()
            for b in range(B):
                lhs = o_ref[pl.ds(b * S + my_base, S_HALF), :]
                rows = slice(b * S_HALF, (b + 1) * S_HALF)
                acc_buf[j % 2, rows, :] = jnp.dot(
                    lhs, wo_buf[j % 2], preferred_element_type=jnp.float32
                ) + recv_buf[j, rows, :].astype(jnp.float32)
            ocps[j] = pltpu.make_async_copy(
                acc_buf.at[j % 2], out_ref.at[:, pl.ds(j * NC, NC)],
                out_sem.at[j % 2])
            ocps[j].start()

        for j in (G - 2, G - 1):
            ocps[j].wait()
        for j in range(G - 4, G):
            rdmas[j].wait_send()

    out = pl.pallas_call(
        body,
        out_shape=jax.ShapeDtypeStruct((ROWS, N), jnp.float32),
        in_specs=[
            pl.BlockSpec(memory_space=pltpu.VMEM),
            pl.BlockSpec(memory_space=pl.ANY),
        ],
        out_specs=pl.BlockSpec(memory_space=pltpu.MemorySpace.HBM),
        scratch_shapes=[
            pltpu.VMEM((2, K, NC), jnp.bfloat16),
            pltpu.VMEM((4, ROWS, NC), jnp.bfloat16),
            pltpu.VMEM((G, ROWS, NC), jnp.bfloat16),
            pltpu.VMEM((2, ROWS, NC), jnp.float32),
            pltpu.SemaphoreType.DMA((2,)),
            pltpu.SemaphoreType.DMA((2,)),
            pltpu.SemaphoreType.DMA((4,)),
            pltpu.SemaphoreType.DMA((G,)),
        ],
        compiler_params=pltpu.CompilerParams(
            collective_id=0, vmem_limit_bytes=56 * 1024 * 1024),
    )(O2, Wo2)
    return out.reshape(B, S_HALF, N)


# device time: 260056 ns/iter; 1.0175x vs baseline; 1.0175x over previous
import jax
import jax.numpy as jnp
from jax import lax
from jax.experimental import pallas as pl
from jax.experimental.pallas import tpu as pltpu

B, S, H_LOC, D = 4, 1024, 16, 128
K = H_LOC * D
N = 4096
S_HALF = S // 2
NC = 512
G = N // NC
ROWS = B * S_HALF


def kernel(O, Wo):
    O2 = O.reshape(B * S, K).astype(jnp.bfloat16)
    Wo2 = Wo.astype(jnp.bfloat16)

    def body(o_ref, wo_hbm, out_ref, wo_buf, send_buf, recv_buf, acc_buf,
             wo_sem, out_sem, send_sems, recv_sems):
        my_x = lax.axis_index("x")
        my_y = lax.axis_index("y")
        my_z = lax.axis_index("z")
        peer = (my_x, 1 - my_y, my_z)

        barrier = pltpu.get_barrier_semaphore()
        pl.semaphore_signal(barrier, inc=1, device_id=peer,
                            device_id_type=pl.DeviceIdType.MESH)
        pl.semaphore_wait(barrier, 1)

        my_base = my_y * S_HALF
        peer_base = (1 - my_y) * S_HALF

        def wo_fetch(j):
            return pltpu.make_async_copy(
                wo_hbm.at[:, pl.ds((j % G) * NC, NC)], wo_buf.at[j % 2],
                wo_sem.at[j % 2])

        rdmas = [None] * G
        ocps = [None] * G

        wo_fetch(0).start()
        for j in range(G):
            if j + 1 < G:
                wo_fetch(j + 1).start()
            wo_fetch(j).wait()
            if j >= 4:
                rdmas[j - 4].wait_send()
            for b in range(B):
                lhs = o_ref[pl.ds(b * S + peer_base, S_HALF), :]
                send_buf[j % 4, b * S_HALF:(b + 1) * S_HALF, :] = jnp.dot(
                    lhs, wo_buf[j % 2], preferred_element_type=jnp.float32
                ).astype(jnp.bfloat16)
            rdmas[j] = pltpu.make_async_remote_copy(
                src_ref=send_buf.at[j % 4],
                dst_ref=recv_buf.at[j],
                send_sem=send_sems.at[j % 4],
                recv_sem=recv_sems.at[j],
                device_id=peer,
                device_id_type=pl.DeviceIdType.MESH,
            )
            rdmas[j].start()

        wo_fetch(G).start()
        for j in range(G):
            if j + 1 < G:
                wo_fetch(G + j + 1).start()
            wo_fetch(G + j).wait()
            if j >= 2:
                ocps[j - 2].wait()
            for b in range(B):
                lhs = o_ref[pl.ds(b * S + my_base, S_HALF), :]
                acc_buf[j % 2, b * S_HALF:(b + 1) * S_HALF, :] = jnp.dot(
                    lhs, wo_buf[j % 2], preferred_element_type=jnp.float32)
            rdmas[j].wait_recv()
            acc_buf[j % 2] = acc_buf[j % 2] + recv_buf[j].astype(jnp.float32)
            ocps[j] = pltpu.make_async_copy(
                acc_buf.at[j % 2], out_ref.at[:, pl.ds(j * NC, NC)],
                out_sem.at[j % 2])
            ocps[j].start()

        for j in (G - 2, G - 1):
            ocps[j].wait()
        for j in range(G - 4, G):
            rdmas[j].wait_send()

    out = pl.pallas_call(
        body,
        out_shape=jax.ShapeDtypeStruct((ROWS, N), jnp.float32),
        in_specs=[
            pl.BlockSpec(memory_space=pltpu.VMEM),
            pl.BlockSpec(memory_space=pl.ANY),
        ],
        out_specs=pl.BlockSpec(memory_space=pltpu.MemorySpace.HBM),
        scratch_shapes=[
            pltpu.VMEM((2, K, NC), jnp.bfloat16),
            pltpu.VMEM((4, ROWS, NC), jnp.bfloat16),
            pltpu.VMEM((G, ROWS, NC), jnp.bfloat16),
            pltpu.VMEM((2, ROWS, NC), jnp.float32),
            pltpu.SemaphoreType.DMA((2,)),
            pltpu.SemaphoreType.DMA((2,)),
            pltpu.SemaphoreType.DMA((4,)),
            pltpu.SemaphoreType.DMA((G,)),
        ],
        compiler_params=pltpu.CompilerParams(
            collective_id=0, vmem_limit_bytes=56 * 1024 * 1024),
    )(O2, Wo2)
    return out.reshape(B, S_HALF, N)
